# baseline (speedup 1.0000x reference)
import jax
import jax.numpy as jnp
from jax.experimental import pallas as pl

NUM_ITEMS = 20000
EMB = 128
NUM_LAYERS = 2
CORE_K = 3


def _propagate(rows, cols, w, feat_a, feat_b):
    a, b = feat_a, feat_b
    acc_a, acc_b = feat_a, feat_b
    for _ in range(NUM_LAYERS):
        na = jnp.zeros_like(a).at[rows].add(w[:, None] * b[cols])
        nb = jnp.zeros_like(b).at[cols].add(w[:, None] * a[rows])
        a, b = na, nb
        acc_a = acc_a + a
        acc_b = acc_b + b
    return acc_a / (NUM_LAYERS + 1.0), acc_b / (NUM_LAYERS + 1.0)


def _combine_kernel(main_ref, syn_ref, out_ref):
    out_ref[...] = main_ref[...] + syn_ref[...]


def kernel(users, bundles, users_feature, bundles_feature, items_feature, ub_rows, ub_cols, ub_w, ui_rows, ui_cols, ui_w, bi_rows, bi_cols, bi_w, bundle_items, w_core1, b_core1, w_core2, b_core2, w_syn1, b_syn1, w_syn2, b_syn2):
    UB_u, UB_b = _propagate(ub_rows, ub_cols, ub_w, users_feature, bundles_feature)
    UI_u, UI_i = _propagate(ui_rows, ui_cols, ui_w, users_feature, items_feature)
    BI_b, BI_i = _propagate(bi_rows, bi_cols, bi_w, bundles_feature, items_feature)
    num_candidates = bundles.shape[1]
    users_expanded = jnp.repeat(users, num_candidates)
    bundles_flat = bundles.reshape(-1)
    bbi = bundle_items[bundles_flat]
    mask = bbi != NUM_ITEMS
    UI_i_pad = jnp.concatenate([UI_i, jnp.zeros((1, EMB), jnp.float32)], axis=0)
    BI_i_pad = jnp.concatenate([BI_i, jnp.zeros((1, EMB), jnp.float32)], axis=0)
    items_ui = UI_i_pad[bbi]
    items_bi = BI_i_pad[bbi]
    u_ui = UI_u[users_expanded]
    b_bi = BI_b[bundles_flat]
    r_ui = jnp.sum(u_ui[:, None, :] * items_ui, axis=2)
    r_bi = jnp.sum(b_bi[:, None, :] * items_bi, axis=2)
    mlp_in = jnp.stack([r_ui, r_bi], axis=2)
    h = jax.nn.relu(mlp_in @ w_core1.T + b_core1)
    core_logits = (h @ w_core2.T + b_core2)[..., 0]
    core_logits = jnp.where(mask, core_logits, -jnp.inf)
    pi = jax.nn.softmax(core_logits, axis=1)
    k = min(CORE_K, bbi.shape[1])
    topk_vals, topk_idx = jax.lax.top_k(pi, k)
    topk_pi = topk_vals / (jnp.sum(topk_vals, axis=1, keepdims=True) + 1e-10)
    core_items = jnp.take_along_axis(items_ui, topk_idx[:, :, None], axis=1)
    h_core = jnp.sum(core_items * topk_pi[:, :, None], axis=1)
    is_core = jnp.zeros(pi.shape, bool).at[jnp.arange(pi.shape[0])[:, None], topk_idx].set(True)
    is_fringe = mask & (~is_core)
    fringe_sum = jnp.sum(items_ui * is_fringe[:, :, None].astype(jnp.float32), axis=1)
    fringe_count = jnp.maximum(jnp.sum(is_fringe, axis=1, keepdims=True).astype(jnp.float32), 1.0)
    h_fringe = fringe_sum / fringe_count
    syn_h = jax.nn.relu(jnp.concatenate([h_core, h_fringe], axis=1) @ w_syn1.T + b_syn1)
    h_syn = syn_h @ w_syn2.T + b_syn2
    synergy = jnp.sum(u_ui * h_syn, axis=1)
    main = jnp.sum(UB_u[users_expanded] * UB_b[bundles_flat], axis=1)
    pred = pl.pallas_call(
        _combine_kernel,
        out_shape=jax.ShapeDtypeStruct((bundles.shape[0], bundles.shape[1]), jnp.float32),
    )(main.reshape(bundles.shape), synergy.reshape(bundles.shape))
    return pred


# speedup vs baseline: 1.3215x; 1.3215x over previous
import functools

import jax
import jax.numpy as jnp
from jax import lax
from jax.experimental import pallas as pl
from jax.experimental.pallas import tpu as pltpu
from jax.experimental.pallas import tpu_sc as plsc

N_USERS = 20000
N_BUNDLES = 8000
N_ITEMS = 20000
EMB = 128
LAYERS = 2
K_CORE = 3

NT = 16
NC = 2
CEDGE = 128
ZR = 8
WR = 8


@functools.lru_cache(maxsize=None)
def _make_prop(n_a, n_b, e_pad, combine):
    ha, hb = n_a // 2, n_b // 2
    maxh = max(ha, hb)
    maxq = max(-(-(ha // NT) // 8) * 8 * NT, -(-(hb // NT) // 8) * 8 * NT)
    maxh = max(maxh, maxq)
    ch = e_pad // (NT * CEDGE)
    mesh = plsc.VectorSubcoreMesh(core_axis_name="c", subcore_axis_name="s")

    out_type = (jax.ShapeDtypeStruct((n_a, EMB), jnp.float32),
                jax.ShapeDtypeStruct((n_b, EMB), jnp.float32))
    scratch = [
        pltpu.VMEM_SHARED((maxh + 1, EMB), jnp.float32),
        pltpu.VMEM((CEDGE,), jnp.int32),
        pltpu.VMEM((CEDGE,), jnp.int32),
        pltpu.VMEM((CEDGE,), jnp.int32),
        pltpu.VMEM((CEDGE,), jnp.int32),
        pltpu.VMEM((CEDGE,), jnp.float32),
        pltpu.VMEM((CEDGE, EMB), jnp.float32),
        pltpu.VMEM((ZR, EMB), jnp.float32),
        pltpu.VMEM((WR, EMB), jnp.float32),
        pltpu.VMEM((WR, EMB), jnp.float32),
        pltpu.VMEM((WR, EMB), jnp.float32),
        pltpu.SemaphoreType.DMA,
    ]

    def body(*refs):
        if combine:
            (a_t, b_t, rows_h, cols_h, w_h, a0_t, b0_t, out_a, out_b,
             shared, sidx, gidx, didx, lidx, wv, rows, zbuf, buf0, buf1, buf2,
             sem) = refs
        else:
            (a_t, b_t, rows_h, cols_h, w_h, out_a, out_b,
             shared, sidx, gidx, didx, lidx, wv, rows, zbuf, buf0, buf1, buf2,
             sem) = refs
            a0_t = b0_t = None
        sid = lax.axis_index("s")
        cid = lax.axis_index("c")

        zv = jnp.zeros((16,), jnp.float32)
        for r in range(ZR):
            for c in range(EMB // 16):
                zbuf[r, pl.ds(c * 16, 16)] = zv

        def run_phase(dst_h, src_h, src_tbl, half, n_src, out_h, prev0, prev1):
            base = cid * half
            share_hi = -(-(half // NT) // 8) * 8
            rem = half - (NT - 1) * share_hi
            my0 = sid * share_hi
            my_rows = jnp.where(sid == NT - 1, rem, share_hi)

            def zb(i, carry):
                @pl.when(i * ZR < my_rows)
                def _():
                    pltpu.sync_copy(zbuf, shared.at[pl.ds(my0 + i * ZR, ZR)])
                return carry
            lax.fori_loop(0, share_hi // ZR, zb, 0)
            plsc.subcore_barrier()

            def eb(ci, carry):
                e0 = (sid * ch + ci) * CEDGE
                pltpu.sync_copy(src_h.at[pl.ds(e0, CEDGE)], sidx)
                pltpu.sync_copy(dst_h.at[pl.ds(e0, CEDGE)], didx)
                pltpu.sync_copy(w_h.at[pl.ds(e0, CEDGE)], wv)
                for q in range(CEDGE // 16):
                    sl = pl.ds(q * 16, 16)
                    gidx[sl] = jnp.minimum(sidx[sl], n_src - 1)
                    d = didx[sl] - base
                    ok = (d >= 0) & (d < half)
                    lidx[sl] = jnp.where(ok, d, half)
                pltpu.async_copy(src_tbl.at[gidx], rows, sem).wait()
                for q in range(CEDGE // 16):
                    w16 = wv[pl.ds(q * 16, 16)]
                    for jj in range(16):
                        j = q * 16 + jj
                        wj = jnp.full((16,), w16[jj])
                        for c in range(EMB // 16):
                            sl = pl.ds(c * 16, 16)
                            rows[j, sl] = rows[j, sl] * wj
                pltpu.sync_copy(rows, shared.at[lidx], add=True)
                return carry
            lax.fori_loop(0, ch, eb, 0)
            plsc.subcore_barrier()

            if not combine:
                @pl.when(sid < NT - 1)
                def _():
                    pltpu.sync_copy(shared.at[pl.ds(my0, share_hi)],
                                    out_h.at[pl.ds(base + my0, share_hi)])

                @pl.when(sid == NT - 1)
                def _():
                    pltpu.sync_copy(shared.at[pl.ds(my0, rem)],
                                    out_h.at[pl.ds(base + my0, rem)])
            else:
                def wb(k, carry):
                    s0 = my0 + k * WR
                    g0 = base + s0

                    @pl.when(k * WR < my_rows)
                    def _():
                        pltpu.sync_copy(prev0.at[pl.ds(g0, WR)], buf0)
                        pltpu.sync_copy(prev1.at[pl.ds(g0, WR)], buf1)
                        pltpu.sync_copy(shared.at[pl.ds(s0, WR)], buf2)
                        for r in range(WR):
                            for c in range(EMB // 16):
                                sl = pl.ds(c * 16, 16)
                                buf0[r, sl] = (buf0[r, sl] + buf1[r, sl]
                                               + buf2[r, sl]) * (1.0 / 3.0)
                        pltpu.sync_copy(buf0, out_h.at[pl.ds(g0, WR)])
                    return carry
                lax.fori_loop(0, share_hi // WR, wb, 0)
            plsc.subcore_barrier()

        run_phase(rows_h, cols_h, b_t, ha, n_b, out_a, a0_t, a_t)
        run_phase(cols_h, rows_h, a_t, hb, n_a, out_b, b0_t, b_t)

    return pl.kernel(body, out_type=out_type, mesh=mesh, scratch_types=scratch)


def _pad_edges(rows, cols, w, n_a, n_b):
    e = rows.shape[0]
    e_pad = -(-e // (NT * CEDGE)) * (NT * CEDGE)
    pad = e_pad - e
    rows_p = jnp.concatenate([rows.astype(jnp.int32),
                              jnp.full((pad,), n_a, jnp.int32)])
    cols_p = jnp.concatenate([cols.astype(jnp.int32),
                              jnp.full((pad,), n_b, jnp.int32)])
    w_p = jnp.concatenate([w, jnp.zeros((pad,), jnp.float32)])
    return rows_p, cols_p, w_p, e_pad


def _propagate_sc(rows, cols, w, feat_a, feat_b):
    n_a, n_b = feat_a.shape[0], feat_b.shape[0]
    rows_p, cols_p, w_p, e_pad = _pad_edges(rows, cols, w, n_a, n_b)
    l1 = _make_prop(n_a, n_b, e_pad, False)
    a1, b1 = l1(feat_a, feat_b, rows_p, cols_p, w_p)
    l2 = _make_prop(n_a, n_b, e_pad, True)
    return l2(a1, b1, rows_p, cols_p, w_p, feat_a, feat_b)


def kernel(users, bundles, users_feature, bundles_feature, items_feature, ub_rows, ub_cols, ub_w, ui_rows, ui_cols, ui_w, bi_rows, bi_cols, bi_w, bundle_items, w_core1, b_core1, w_core2, b_core2, w_syn1, b_syn1, w_syn2, b_syn2):
    UB_u, UB_b = _propagate_sc(ub_rows, ub_cols, ub_w, users_feature, bundles_feature)
    UI_u, UI_i = _propagate_sc(ui_rows, ui_cols, ui_w, users_feature, items_feature)
    BI_b, BI_i = _propagate_sc(bi_rows, bi_cols, bi_w, bundles_feature, items_feature)
    num_candidates = bundles.shape[1]
    users_expanded = jnp.repeat(users, num_candidates)
    bundles_flat = bundles.reshape(-1)
    bbi = bundle_items[bundles_flat]
    mask = bbi != N_ITEMS
    UI_i_pad = jnp.concatenate([UI_i, jnp.zeros((1, EMB), jnp.float32)], axis=0)
    BI_i_pad = jnp.concatenate([BI_i, jnp.zeros((1, EMB), jnp.float32)], axis=0)
    items_ui = UI_i_pad[bbi]
    items_bi = BI_i_pad[bbi]
    u_ui = UI_u[users_expanded]
    b_bi = BI_b[bundles_flat]
    r_ui = jnp.sum(u_ui[:, None, :] * items_ui, axis=2)
    r_bi = jnp.sum(b_bi[:, None, :] * items_bi, axis=2)
    mlp_in = jnp.stack([r_ui, r_bi], axis=2)
    h = jax.nn.relu(mlp_in @ w_core1.T + b_core1)
    core_logits = (h @ w_core2.T + b_core2)[..., 0]
    core_logits = jnp.where(mask, core_logits, -jnp.inf)
    pi = jax.nn.softmax(core_logits, axis=1)
    k = min(K_CORE, bbi.shape[1])
    topk_vals, topk_idx = jax.lax.top_k(pi, k)
    topk_pi = topk_vals / (jnp.sum(topk_vals, axis=1, keepdims=True) + 1e-10)
    core_items = jnp.take_along_axis(items_ui, topk_idx[:, :, None], axis=1)
    h_core = jnp.sum(core_items * topk_pi[:, :, None], axis=1)
    is_core = jnp.zeros(pi.shape, bool).at[jnp.arange(pi.shape[0])[:, None], topk_idx].set(True)
    is_fringe = mask & (~is_core)
    fringe_sum = jnp.sum(items_ui * is_fringe[:, :, None].astype(jnp.float32), axis=1)
    fringe_count = jnp.maximum(jnp.sum(is_fringe, axis=1, keepdims=True).astype(jnp.float32), 1.0)
    h_fringe = fringe_sum / fringe_count
    syn_h = jax.nn.relu(jnp.concatenate([h_core, h_fringe], axis=1) @ w_syn1.T + b_syn1)
    h_syn = syn_h @ w_syn2.T + b_syn2
    synergy = jnp.sum(u_ui * h_syn, axis=1)
    main = jnp.sum(UB_u[users_expanded] * UB_b[bundles_flat], axis=1)
    return (main + synergy).reshape(bundles.shape)
